# Initial kernel scaffold; baseline (speedup 1.0000x reference)
import jax
import jax.numpy as jnp
from jax.experimental import pallas as pl


def kernel(src_feats, src_points, query_feats, query_points, gauss_B, cg, cb, Wq, Wkv, Wo, bo, fg, fb, W1, b1, W2, b2):
    raise NotImplementedError("write your pallas kernel here")


# speedup vs baseline: 23.1549x; 23.1549x over previous
import functools
import math

import jax
import jax.numpy as jnp
from jax import lax
from jax.experimental import pallas as pl
from jax.experimental.pallas import tpu as pltpu
from jax.experimental.pallas import tpu_sc as plsc

DIM = 128
H = 4
DH = 64
MLP = 256
DEPTH = 2
K = 16
NS = 20000
NQ = 4096
NSP = 20096
QB = 128
YB = QB * K
F32 = jnp.float32
TWO_PI = 2.0 * math.pi
SCALE = DH ** (-0.5)


def _ln(v, g, b):
    mu = jnp.mean(v, axis=-1, keepdims=True)
    var = jnp.mean((v - mu) ** 2, axis=-1, keepdims=True)
    return (v - mu) / jnp.sqrt(var + 1e-5) * g + b


def _gelu(v):
    return 0.5 * v * (1.0 + jnp.tanh(math.sqrt(2.0 / math.pi) * (v + 0.044715 * v ** 3)))


def _table_body(sf_ref, sps_ref, g_ref, out_ref):
    proj = jnp.dot(sps_ref[...], g_ref[...], preferred_element_type=F32)
    out_ref[:, :DIM] = sf_ref[...]
    out_ref[:, DIM:DIM + DH * 2] = jnp.concatenate(
        [jnp.sin(proj), jnp.cos(proj)], axis=1)


def _build_table(sf_pad, sps_pad, g_pad):
    br = 1256
    grid = NSP // br
    return pl.pallas_call(
        _table_body,
        grid=(grid,),
        in_specs=[
            pl.BlockSpec((br, DIM), lambda i: (i, 0)),
            pl.BlockSpec((br, DIM), lambda i: (i, 0)),
            pl.BlockSpec((DIM, DH), lambda i: (0, 0)),
        ],
        out_specs=pl.BlockSpec((br, 2 * DIM), lambda i: (i, 0)),
        out_shape=jax.ShapeDtypeStruct((NSP, 2 * DIM), F32),
    )(sf_pad, sps_pad, g_pad)


def _topk_body(qp_ref, spt_ref, out_ref):
    qp = qp_ref[...]
    d = jnp.zeros((QB, NSP), F32)
    dx = qp[:, 0:1] - spt_ref[0:1, :]
    dy = qp[:, 1:2] - spt_ref[1:2, :]
    dz = qp[:, 2:3] - spt_ref[2:3, :]
    d = (dx * dx + dy * dy) + dz * dz
    iota = lax.broadcasted_iota(jnp.int32, (QB, NSP), 1).astype(F32)
    col = lax.broadcasted_iota(jnp.int32, (QB, DIM), 1).astype(F32)
    idxs = jnp.zeros((QB, DIM), F32)
    for t in range(K):
        m = jnp.min(d, axis=1, keepdims=True)
        am = jnp.min(jnp.where(d <= m, iota, F32(3e7)), axis=1, keepdims=True)
        d = jnp.where(iota == am, F32(jnp.inf), d)
        idxs = jnp.where(col == F32(t), am, idxs)
    out_ref[...] = idxs.astype(jnp.int32)


def _topk(qp_pad, spt_pad):
    return pl.pallas_call(
        _topk_body,
        grid=(NQ // QB,),
        in_specs=[
            pl.BlockSpec((QB, DIM), lambda i: (i, 0)),
            pl.BlockSpec((DIM, NSP), lambda i: (0, 0)),
        ],
        out_specs=pl.BlockSpec((QB, DIM), lambda i: (i, 0)),
        out_shape=jax.ShapeDtypeStruct((NQ, DIM), jnp.int32),
    )(qp_pad, spt_pad)


def _sc_gather(table, flat_idx):
    n_idx = NQ * K
    nw = 32
    b_per_w = n_idx // nw
    chunk = 256
    mesh = plsc.VectorSubcoreMesh(core_axis_name="c", subcore_axis_name="s")

    @functools.partial(
        pl.kernel,
        out_type=jax.ShapeDtypeStruct((n_idx, 2 * DIM), F32),
        mesh=mesh,
        scratch_types=[
            pltpu.VMEM((b_per_w,), jnp.int32),
            pltpu.VMEM((chunk, 2 * DIM), F32),
            pltpu.SemaphoreType.DMA,
        ],
    )
    def gather_kernel(table_hbm, idx_hbm, out_hbm, idx_v, rows_v, sem):
        wid = lax.axis_index("s") * 2 + lax.axis_index("c")
        base = wid * b_per_w
        pltpu.sync_copy(idx_hbm.at[pl.ds(base, b_per_w)], idx_v)
        for c in range(b_per_w // chunk):
            pltpu.async_copy(
                table_hbm.at[idx_v.at[pl.ds(c * chunk, chunk)]], rows_v, sem
            ).wait()
            pltpu.sync_copy(rows_v, out_hbm.at[pl.ds(base + c * chunk, chunk)])

    return gather_kernel(table, flat_idx)


def _tf_body(qf_ref, qps_ref, g_ref, gath_ref,
             cg_ref, cb_ref, wq_ref, wkv_ref, wo_ref, bo_ref,
             fg_ref, fb_ref, w1_ref, b1_ref, w2_ref, b2_ref, out_ref):
    x = qf_ref[...]
    proj = jnp.dot(qps_ref[...], g_ref[...], preferred_element_type=F32)
    qpos = jnp.concatenate([jnp.sin(proj), jnp.cos(proj)], axis=1)
    g = gath_ref[...]
    yb = g[:, :DIM] + g[:, DIM:]

    seg = (lax.broadcasted_iota(jnp.int32, (2 * DIM, DIM), 0) // DH
           == lax.broadcasted_iota(jnp.int32, (2 * DIM, DIM), 1)).astype(F32)
    expand = (lax.broadcasted_iota(jnp.int32, (DIM, 2 * DIM), 0)
              == lax.broadcasted_iota(jnp.int32, (DIM, 2 * DIM), 1) // DH
              ).astype(F32)

    for i in range(DEPTH):
        xin = _ln(x + qpos, cg_ref[i], cb_ref[i])
        yin = _ln(yb, cg_ref[i], cb_ref[i])
        q = jnp.dot(xin, wq_ref[i], preferred_element_type=F32)
        kv = jnp.dot(yin, wkv_ref[i], preferred_element_type=F32)
        k = kv[:, :H * DH]
        v = kv[:, H * DH:]
        qb = jnp.broadcast_to(
            q.reshape(QB, 1, H * DH), (QB, K, H * DH)).reshape(YB, H * DH)
        dots = jnp.dot(k * qb, seg, preferred_element_type=F32) * SCALE
        dots3 = dots.reshape(QB, K, DIM)
        mx = jnp.max(dots3, axis=1, keepdims=True)
        e = jnp.exp(dots3 - mx)
        attn = e / jnp.sum(e, axis=1, keepdims=True)
        a = jnp.dot(attn.reshape(YB, DIM), expand, preferred_element_type=F32)
        o = (a * v).reshape(QB, K, H * DH).sum(axis=1)
        x = jnp.dot(o, wo_ref[i], preferred_element_type=F32) + bo_ref[i] + x
        h2 = _ln(x, fg_ref[i], fb_ref[i])
        m = jnp.dot(h2, w1_ref[i], preferred_element_type=F32) + b1_ref[i]
        x = jnp.dot(_gelu(m), w2_ref[i], preferred_element_type=F32) + b2_ref[i] + x
    out_ref[...] = x


def _transformer(qf, qps_pad, g_pad, gathered,
                 cg, cb, wq, wkv, wo, bo, fg, fb, w1, b1, w2, b2):
    full = lambda *shape: pl.BlockSpec(shape, lambda i: (0,) * len(shape))
    return pl.pallas_call(
        _tf_body,
        grid=(NQ // QB,),
        in_specs=[
            pl.BlockSpec((QB, DIM), lambda i: (i, 0)),
            pl.BlockSpec((QB, DIM), lambda i: (i, 0)),
            full(DIM, DH),
            pl.BlockSpec((YB, 2 * DIM), lambda i: (i, 0)),
            full(DEPTH, DIM), full(DEPTH, DIM),
            full(DEPTH, DIM, H * DH), full(DEPTH, DIM, 2 * H * DH),
            full(DEPTH, H * DH, DIM), full(DEPTH, DIM),
            full(DEPTH, DIM), full(DEPTH, DIM),
            full(DEPTH, DIM, MLP), full(DEPTH, MLP),
            full(DEPTH, MLP, DIM), full(DEPTH, DIM),
        ],
        out_specs=pl.BlockSpec((QB, DIM), lambda i: (i, 0)),
        out_shape=jax.ShapeDtypeStruct((NQ, DIM), F32),
    )(qf, qps_pad, g_pad, gathered,
      cg, cb, wq, wkv, wo, bo, fg, fb, w1, b1, w2, b2)


def kernel(src_feats, src_points, query_feats, query_points, gauss_B,
           cg, cb, Wq, Wkv, Wo, bo, fg, fb, W1, b1, W2, b2):
    pad_rows = NSP - NS
    sf_pad = jnp.pad(src_feats, ((0, pad_rows), (0, 0)))
    sps_pad = jnp.pad(src_points * TWO_PI, ((0, pad_rows), (0, DIM - 3)))
    g_pad = jnp.pad(gauss_B, ((0, DIM - 3), (0, 0)))
    qps_pad = jnp.pad(query_points * TWO_PI, ((0, 0), (0, DIM - 3)))
    spc = jnp.pad(src_points, ((0, pad_rows), (0, 0)), constant_values=1e3)
    spt_pad = jnp.pad(spc.T, ((0, DIM - 3), (0, 0)))
    qp_pad = jnp.pad(query_points, ((0, 0), (0, DIM - 3)))

    knn_idx = _topk(qp_pad, spt_pad)[:, :K]
    table = _build_table(sf_pad, sps_pad, g_pad)
    gathered = _sc_gather(table, knn_idx.reshape(-1))
    return _transformer(query_feats, qps_pad, g_pad, gathered,
                        cg, cb, Wq, Wkv, Wo, bo, fg, fb, W1, b1, W2, b2)
